# Initial kernel scaffold; baseline (speedup 1.0000x reference)
import jax
import jax.numpy as jnp
from jax.experimental import pallas as pl


def kernel(x, edge_index, edge_attr, batch, W, att_src, att_dst, W_edge, att_edge, bias012, bias3, gamma012, beta012, gamma3, beta3):
    raise NotImplementedError("write your pallas kernel here")


# speedup vs baseline: 30.6530x; 30.6530x over previous
import functools

import jax
import jax.numpy as jnp
from jax import lax
from jax.experimental import pallas as pl
from jax.experimental.pallas import tpu as pltpu
from jax.experimental.pallas import tpu_sc as plsc

N = 10000
E = 320000
D = 128
NH = 8
C = 16
NG = 64
NL = 4
HP = 16

NC = 2
NS = 16
NW = NC * NS

NPAD = 10240
RPT = NPAD // NS
EPT = E // NW
EB = 80
NCHUNK = EPT // EB

BN = 2000
NB = N // BN
BE = 4000
f32 = jnp.float32

_MESH = plsc.VectorSubcoreMesh(core_axis_name="c", subcore_axis_name="s")


def _sc_attn_body(src_hbm, dst_hbm, ae_hbm, as_hbm, ad_hbm, z16_hbm,
                  ex_hbm, dp_hbm,
                  denom_sh, sidx_v, didx_v, asg_v, adg_v, aeg_v, exv_v, sem):
    c = lax.axis_index("c")
    s = lax.axis_index("s")
    wid = s * NC + c
    r0 = s * RPT
    pltpu.sync_copy(z16_hbm.at[pl.ds(r0, RPT), :],
                    denom_sh.at[pl.ds(r0, RPT), :])
    plsc.subcore_barrier()
    ebase = wid * EPT

    def chunk(it, carry):
        off = ebase + it * EB
        pltpu.sync_copy(src_hbm.at[pl.ds(off, EB)], sidx_v)
        pltpu.sync_copy(dst_hbm.at[pl.ds(off, EB)], didx_v)
        pltpu.async_copy(as_hbm.at[sidx_v], asg_v, sem).wait()
        pltpu.async_copy(ad_hbm.at[didx_v], adg_v, sem).wait()
        pltpu.sync_copy(ae_hbm.at[pl.ds(off, EB), :], aeg_v)

        def elt(e, c2):
            v = asg_v[e, :] + adg_v[e, :] + aeg_v[e, :]
            v = jnp.where(v >= 0.0, v, 0.2 * v)
            exv_v[e, :] = jnp.exp(v)
            return c2

        lax.fori_loop(0, EB, elt, 0)
        pltpu.sync_copy(exv_v, ex_hbm.at[pl.ds(off, EB), :])
        pltpu.sync_copy(exv_v, denom_sh.at[didx_v], add=True)
        return carry

    lax.fori_loop(0, NCHUNK, chunk, 0)
    plsc.subcore_barrier()
    pltpu.sync_copy(denom_sh.at[pl.ds(r0, RPT), :],
                    dp_hbm.at[c, pl.ds(r0, RPT), :])


_SC_PARAMS = pltpu.CompilerParams(use_tc_tiling_on_sc=False)

_sc_attn = functools.partial(
    pl.kernel, mesh=_MESH, compiler_params=_SC_PARAMS,
    out_type=(jax.ShapeDtypeStruct((E, HP), f32),
              jax.ShapeDtypeStruct((2, NPAD, HP), f32)),
    scratch_types=[
        pltpu.VMEM_SHARED((NPAD, HP), f32),
        pltpu.VMEM((EB,), jnp.int32), pltpu.VMEM((EB,), jnp.int32),
        pltpu.VMEM((EB, HP), f32), pltpu.VMEM((EB, HP), f32),
        pltpu.VMEM((EB, HP), f32), pltpu.VMEM((EB, HP), f32),
        pltpu.SemaphoreType.DMA,
    ])(_sc_attn_body)


def _sc_msg_body(src_hbm, dst_hbm, ex_hbm, d0_hbm, d1_hbm, h_hbm, z128_hbm,
                 op_hbm,
                 out_sh, sidx_v, didx_v, exv_v, d0g_v, d1g_v, rows_v, sem):
    c = lax.axis_index("c")
    s = lax.axis_index("s")
    wid = s * NC + c
    r0 = s * RPT
    pltpu.sync_copy(z128_hbm.at[pl.ds(r0, RPT), :],
                    out_sh.at[pl.ds(r0, RPT), :])
    plsc.subcore_barrier()
    ebase = wid * EPT

    def chunk(it, carry):
        off = ebase + it * EB
        pltpu.sync_copy(src_hbm.at[pl.ds(off, EB)], sidx_v)
        pltpu.sync_copy(dst_hbm.at[pl.ds(off, EB)], didx_v)
        pltpu.async_copy(h_hbm.at[sidx_v], rows_v, sem).wait()
        pltpu.sync_copy(ex_hbm.at[pl.ds(off, EB), :], exv_v)
        pltpu.async_copy(d0_hbm.at[didx_v], d0g_v, sem).wait()
        pltpu.async_copy(d1_hbm.at[didx_v], d1g_v, sem).wait()

        def scale(e, c2):
            dv = d0g_v[e, :] + d1g_v[e, :]
            att = exv_v[e, :] / (dv + 1e-16)
            for hh in range(NH):
                sc = att[hh]
                rows_v[e, pl.ds(hh * C, C)] = rows_v[e, pl.ds(hh * C, C)] * sc
            return c2

        lax.fori_loop(0, EB, scale, 0)
        pltpu.sync_copy(rows_v, out_sh.at[didx_v], add=True)
        return carry

    lax.fori_loop(0, NCHUNK, chunk, 0)
    plsc.subcore_barrier()
    pltpu.sync_copy(out_sh.at[pl.ds(r0, RPT), :],
                    op_hbm.at[c, pl.ds(r0, RPT), :])


_sc_msg = functools.partial(
    pl.kernel, mesh=_MESH, compiler_params=_SC_PARAMS,
    out_type=jax.ShapeDtypeStruct((2, NPAD, 128), f32),
    scratch_types=[
        pltpu.VMEM_SHARED((NPAD, 128), f32),
        pltpu.VMEM((EB,), jnp.int32), pltpu.VMEM((EB,), jnp.int32),
        pltpu.VMEM((EB, HP), f32), pltpu.VMEM((EB, HP), f32),
        pltpu.VMEM((EB, HP), f32), pltpu.VMEM((EB, 128), f32),
        pltpu.SemaphoreType.DMA,
    ])(_sc_msg_body)


def _mm_proj_body(x_ref, w_ref, as_ref, ad_ref, h_ref, asg_ref, adg_ref):
    h = jnp.dot(x_ref[...], w_ref[...], preferred_element_type=f32)
    h_ref[...] = h
    asg_ref[...] = jnp.dot(h, as_ref[...], preferred_element_type=f32)
    adg_ref[...] = jnp.dot(h, ad_ref[...], preferred_element_type=f32)


def _mm_proj(x, w, a_s, a_d):
    return pl.pallas_call(
        _mm_proj_body,
        grid=(NB,),
        in_specs=[pl.BlockSpec((BN, D), lambda i: (i, 0)),
                  pl.BlockSpec((D, D), lambda i: (0, 0)),
                  pl.BlockSpec((D, HP), lambda i: (0, 0)),
                  pl.BlockSpec((D, HP), lambda i: (0, 0))],
        out_specs=[pl.BlockSpec((BN, D), lambda i: (i, 0)),
                   pl.BlockSpec((BN, HP), lambda i: (i, 0)),
                   pl.BlockSpec((BN, HP), lambda i: (i, 0))],
        out_shape=[jax.ShapeDtypeStruct((N, D), f32),
                   jax.ShapeDtypeStruct((N, HP), f32),
                   jax.ShapeDtypeStruct((N, HP), f32)],
    )(x, w, a_s, a_d)


def _ae_mm_body(ea_ref, ve_ref, o0, o1, o2, o3):
    r = jnp.dot(ea_ref[...], ve_ref[...], preferred_element_type=f32)
    o0[...] = r[:, 0:HP]
    o1[...] = r[:, HP:2 * HP]
    o2[...] = r[:, 2 * HP:3 * HP]
    o3[...] = r[:, 3 * HP:4 * HP]


def _ae_mm(edge_attr, vec):
    eout = pl.BlockSpec((BE, HP), lambda i: (i, 0))
    return pl.pallas_call(
        _ae_mm_body,
        grid=(E // BE,),
        in_specs=[pl.BlockSpec((BE, D), lambda i: (i, 0)),
                  pl.BlockSpec((D, NL * HP), lambda i: (0, 0))],
        out_specs=[eout, eout, eout, eout],
        out_shape=[jax.ShapeDtypeStruct((E, HP), f32)] * NL,
    )(edge_attr, vec)


def _dense_mid_body(p_ref, bias_ref, g_ref, be_ref, w_ref, as_ref, ad_ref,
                    h_ref, asg_ref, adg_ref, sum_ref, ssq_ref):
    ph = pl.program_id(0)
    bi = pl.program_id(1)
    o = p_ref[0] + p_ref[1] + bias_ref[...]

    @pl.when(jnp.logical_and(ph == 0, bi == 0))
    def _():
        sum_ref[...] = jnp.zeros_like(sum_ref)
        ssq_ref[...] = jnp.zeros_like(ssq_ref)

    @pl.when(ph == 0)
    def _():
        sum_ref[...] += o.sum(axis=0, keepdims=True)
        ssq_ref[...] += (o * o).sum(axis=0, keepdims=True)

    @pl.when(ph == 1)
    def _():
        mean = sum_ref[...] / N
        var = ssq_ref[...] / N - mean * mean
        xn = (o - mean) * lax.rsqrt(var + 1e-5) * g_ref[...] + be_ref[...]
        a = jnp.where(xn > 0, xn, jnp.exp(xn) - 1.0)
        h = jnp.dot(a, w_ref[...], preferred_element_type=f32)
        h_ref[...] = h
        asg_ref[...] = jnp.dot(h, as_ref[...], preferred_element_type=f32)
        adg_ref[...] = jnp.dot(h, ad_ref[...], preferred_element_type=f32)


def _dense_mid(p, bias, g, be, w, a_s, a_d):
    vec = pl.BlockSpec((1, D), lambda ph, i: (0, 0))
    return pl.pallas_call(
        _dense_mid_body,
        grid=(2, NB),
        in_specs=[pl.BlockSpec((2, BN, D), lambda ph, i: (0, i, 0)),
                  vec, vec, vec,
                  pl.BlockSpec((D, D), lambda ph, i: (0, 0)),
                  pl.BlockSpec((D, HP), lambda ph, i: (0, 0)),
                  pl.BlockSpec((D, HP), lambda ph, i: (0, 0))],
        out_specs=[pl.BlockSpec((BN, D), lambda ph, i: (i, 0)),
                   pl.BlockSpec((BN, HP), lambda ph, i: (i, 0)),
                   pl.BlockSpec((BN, HP), lambda ph, i: (i, 0))],
        out_shape=[jax.ShapeDtypeStruct((N, D), f32),
                   jax.ShapeDtypeStruct((N, HP), f32),
                   jax.ShapeDtypeStruct((N, HP), f32)],
        scratch_shapes=[pltpu.VMEM((1, D), f32), pltpu.VMEM((1, D), f32)],
    )(p, bias, g, be, w, a_s, a_d)


def _dense_final_body(p_ref, b3_ref, g3_ref, be3_ref, batch_ref,
                      out_ref, sum_ref, ssq_ref, pool_ref, cnt_ref):
    ph = pl.program_id(0)
    bi = pl.program_id(1)
    o = p_ref[0] + p_ref[1]
    om = o[:, 0:C]
    for hh in range(1, NH):
        om = om + o[:, hh * C:(hh + 1) * C]
    om = om * (1.0 / NH) + b3_ref[...]

    @pl.when(jnp.logical_and(ph == 0, bi == 0))
    def _():
        sum_ref[...] = jnp.zeros_like(sum_ref)
        ssq_ref[...] = jnp.zeros_like(ssq_ref)
        pool_ref[...] = jnp.zeros_like(pool_ref)
        cnt_ref[...] = jnp.zeros_like(cnt_ref)

    @pl.when(ph == 0)
    def _():
        sum_ref[...] += om.sum(axis=0, keepdims=True)
        ssq_ref[...] += (om * om).sum(axis=0, keepdims=True)

    @pl.when(ph == 1)
    def _():
        mean = sum_ref[...] / N
        var = ssq_ref[...] / N - mean * mean
        xn = (om - mean) * lax.rsqrt(var + 1e-5) * g3_ref[...] + be3_ref[...]
        a = jnp.where(xn > 0, xn, jnp.exp(xn) - 1.0)
        ids = lax.broadcasted_iota(jnp.int32, (BN, NG), 1)
        pf = (batch_ref[...] == ids).astype(f32)
        dn = (((0,), (0,)), ((), ()))
        pool_ref[...] += lax.dot_general(pf, a, dn, preferred_element_type=f32)
        cnt_ref[...] += lax.dot_general(pf, jnp.ones((BN, 1), f32), dn,
                                        preferred_element_type=f32)

        @pl.when(bi == NB - 1)
        def _():
            out_ref[...] = pool_ref[...] / jnp.maximum(cnt_ref[...], 1.0)


def _dense_final(p, b3, g3, be3, batch2d):
    vec = pl.BlockSpec((1, C), lambda ph, i: (0, 0))
    return pl.pallas_call(
        _dense_final_body,
        grid=(2, NB),
        in_specs=[pl.BlockSpec((2, BN, D), lambda ph, i: (0, i, 0)),
                  vec, vec, vec,
                  pl.BlockSpec((BN, 1), lambda ph, i: (i, 0))],
        out_specs=pl.BlockSpec((NG, C), lambda ph, i: (0, 0)),
        out_shape=jax.ShapeDtypeStruct((NG, C), f32),
        scratch_shapes=[pltpu.VMEM((1, C), f32), pltpu.VMEM((1, C), f32),
                        pltpu.VMEM((NG, C), f32), pltpu.VMEM((NG, 1), f32)],
    )(p, b3, g3, be3, batch2d)


def kernel(x, edge_index, edge_attr, batch, W, att_src, att_dst, W_edge,
           att_edge, bias012, bias3, gamma012, beta012, gamma3, beta3):
    src = edge_index[0]
    dst = edge_index[1]
    hc = jnp.arange(D)
    heads = jnp.arange(NH)
    blkmask = (hc[:, None] // C == heads[None, :]).astype(f32)
    zpad = jnp.zeros((NL, D, HP - NH), f32)
    a_s = jnp.concatenate(
        [blkmask[None] * att_src.reshape(NL, D, 1), zpad], axis=2)
    a_d = jnp.concatenate(
        [blkmask[None] * att_dst.reshape(NL, D, 1), zpad], axis=2)
    ve = (W_edge.reshape(NL, D, NH, C) * att_edge[:, None]).sum(-1)
    ve = jnp.concatenate([ve, jnp.zeros((NL, D, HP - NH), f32)], axis=2)
    vec = ve.transpose(1, 0, 2).reshape(D, NL * HP)

    ae_all = _ae_mm(edge_attr, vec)
    z16 = jnp.zeros((NPAD, HP), f32)
    z128 = jnp.zeros((NPAD, 128), f32)

    h, asg, adg = _mm_proj(x, W[0], a_s[0], a_d[0])
    out = None
    for i in range(NL):
        ex, dp = _sc_attn(src, dst, ae_all[i], asg, adg, z16)
        op = _sc_msg(src, dst, ex, dp[0], dp[1], h, z128)
        if i < NL - 1:
            h, asg, adg = _dense_mid(op, bias012[i].reshape(1, D),
                                     gamma012[i].reshape(1, D),
                                     beta012[i].reshape(1, D),
                                     W[i + 1], a_s[i + 1], a_d[i + 1])
        else:
            out = _dense_final(op, bias3.reshape(1, C), gamma3.reshape(1, C),
                               beta3.reshape(1, C),
                               batch.reshape(N, 1).astype(jnp.int32))
    return out


# speedup vs baseline: 48.8715x; 1.5943x over previous
import functools

import jax
import jax.numpy as jnp
from jax import lax
from jax.experimental import pallas as pl
from jax.experimental.pallas import tpu as pltpu
from jax.experimental.pallas import tpu_sc as plsc

N = 10000
E = 320000
D = 128
NH = 8
C = 16
NG = 64
NL = 4
HP = 16

NC = 2
NS = 16
NW = NC * NS

NPAD = 10240
RPT = NPAD // NS
EPT = E // NW
EB = 80
NCHUNK = EPT // EB

BN = 2000
NB = N // BN
BE = 4000
f32 = jnp.float32

_MESH = plsc.VectorSubcoreMesh(core_axis_name="c", subcore_axis_name="s")


def _sc_attn_body(src_hbm, dst_hbm, ae_hbm, as_hbm, ad_hbm, z16_hbm,
                  ex_hbm, dp_hbm,
                  denom_sh, sidx_v, didx_v, asg_v, adg_v, aeg_v, exv_v,
                  sem, sem2):
    c = lax.axis_index("c")
    s = lax.axis_index("s")
    wid = s * NC + c
    r0 = s * RPT
    pltpu.sync_copy(z16_hbm.at[pl.ds(r0, RPT), :],
                    denom_sh.at[pl.ds(r0, RPT), :])
    plsc.subcore_barrier()
    ebase = wid * EPT

    def chunk(it, carry):
        off = ebase + it * EB
        ci1 = pltpu.async_copy(src_hbm.at[pl.ds(off, EB)], sidx_v, sem)
        ci2 = pltpu.async_copy(dst_hbm.at[pl.ds(off, EB)], didx_v, sem)
        ce = pltpu.async_copy(ae_hbm.at[pl.ds(off, EB), :], aeg_v, sem2)
        ci1.wait()
        ci2.wait()
        g1 = pltpu.async_copy(as_hbm.at[sidx_v], asg_v, sem)
        g2 = pltpu.async_copy(ad_hbm.at[didx_v], adg_v, sem)
        g1.wait()
        g2.wait()
        ce.wait()

        def elt(e, c2):
            v = asg_v[e, :] + adg_v[e, :] + aeg_v[e, :]
            v = jnp.where(v >= 0.0, v, 0.2 * v)
            exv_v[e, :] = jnp.exp(v)
            return c2

        lax.fori_loop(0, EB, elt, 0)
        pltpu.sync_copy(exv_v, ex_hbm.at[pl.ds(off, EB), :])
        pltpu.sync_copy(exv_v, denom_sh.at[didx_v], add=True)
        return carry

    lax.fori_loop(0, NCHUNK, chunk, 0)
    plsc.subcore_barrier()
    pltpu.sync_copy(denom_sh.at[pl.ds(r0, RPT), :],
                    dp_hbm.at[c, pl.ds(r0, RPT), :])


_SC_PARAMS = pltpu.CompilerParams(use_tc_tiling_on_sc=False)

_sc_attn = functools.partial(
    pl.kernel, mesh=_MESH, compiler_params=_SC_PARAMS,
    out_type=(jax.ShapeDtypeStruct((E, HP), f32),
              jax.ShapeDtypeStruct((2, NPAD, HP), f32)),
    scratch_types=[
        pltpu.VMEM_SHARED((NPAD, HP), f32),
        pltpu.VMEM((EB,), jnp.int32), pltpu.VMEM((EB,), jnp.int32),
        pltpu.VMEM((EB, HP), f32), pltpu.VMEM((EB, HP), f32),
        pltpu.VMEM((EB, HP), f32), pltpu.VMEM((EB, HP), f32),
        pltpu.SemaphoreType.DMA, pltpu.SemaphoreType.DMA,
    ])(_sc_attn_body)


def _sc_msg_body(src_hbm, dst_hbm, ex_hbm, d0_hbm, d1_hbm, h_hbm, z128_hbm,
                 op_hbm,
                 out_sh, dcomb_sh, sidx_v, didx_v, exv_v, d0t_v, d1t_v,
                 dg_v, rows_v, sem, sem2):
    c = lax.axis_index("c")
    s = lax.axis_index("s")
    wid = s * NC + c
    r0 = s * RPT
    pltpu.sync_copy(z128_hbm.at[pl.ds(r0, RPT), :],
                    out_sh.at[pl.ds(r0, RPT), :])
    pltpu.sync_copy(d0_hbm.at[pl.ds(r0, RPT), :], d0t_v)
    pltpu.sync_copy(d1_hbm.at[pl.ds(r0, RPT), :], d1t_v)

    def comb(rr, c2):
        d0t_v[rr, :] = d0t_v[rr, :] + d1t_v[rr, :]
        return c2

    lax.fori_loop(0, RPT, comb, 0)
    pltpu.sync_copy(d0t_v, dcomb_sh.at[pl.ds(r0, RPT), :])
    plsc.subcore_barrier()
    ebase = wid * EPT

    def chunk(it, carry):
        off = ebase + it * EB
        ci1 = pltpu.async_copy(src_hbm.at[pl.ds(off, EB)], sidx_v, sem)
        ci2 = pltpu.async_copy(dst_hbm.at[pl.ds(off, EB)], didx_v, sem)
        ce = pltpu.async_copy(ex_hbm.at[pl.ds(off, EB), :], exv_v, sem2)
        ci1.wait()
        ci2.wait()
        gh = pltpu.async_copy(h_hbm.at[sidx_v], rows_v, sem)
        gd = pltpu.async_copy(dcomb_sh.at[didx_v], dg_v, sem2)
        ce.wait()
        gd.wait()
        gh.wait()

        def scale(e, c2):
            att = exv_v[e, :] / (dg_v[e, :] + 1e-16)
            for hh in range(NH):
                sc = att[hh]
                rows_v[e, pl.ds(hh * C, C)] = rows_v[e, pl.ds(hh * C, C)] * sc
            return c2

        lax.fori_loop(0, EB, scale, 0)
        pltpu.sync_copy(rows_v, out_sh.at[didx_v], add=True)
        return carry

    lax.fori_loop(0, NCHUNK, chunk, 0)
    plsc.subcore_barrier()
    pltpu.sync_copy(out_sh.at[pl.ds(r0, RPT), :],
                    op_hbm.at[c, pl.ds(r0, RPT), :])


_sc_msg = functools.partial(
    pl.kernel, mesh=_MESH, compiler_params=_SC_PARAMS,
    out_type=jax.ShapeDtypeStruct((2, NPAD, 128), f32),
    scratch_types=[
        pltpu.VMEM_SHARED((NPAD, 128), f32),
        pltpu.VMEM_SHARED((NPAD, HP), f32),
        pltpu.VMEM((EB,), jnp.int32), pltpu.VMEM((EB,), jnp.int32),
        pltpu.VMEM((EB, HP), f32),
        pltpu.VMEM((RPT, HP), f32), pltpu.VMEM((RPT, HP), f32),
        pltpu.VMEM((EB, HP), f32), pltpu.VMEM((EB, 128), f32),
        pltpu.SemaphoreType.DMA, pltpu.SemaphoreType.DMA,
    ])(_sc_msg_body)


def _mm_proj_body(x_ref, w_ref, as_ref, ad_ref, h_ref, asg_ref, adg_ref):
    h = jnp.dot(x_ref[...], w_ref[...], preferred_element_type=f32)
    h_ref[...] = h
    asg_ref[...] = jnp.dot(h, as_ref[...], preferred_element_type=f32)
    adg_ref[...] = jnp.dot(h, ad_ref[...], preferred_element_type=f32)


def _mm_proj(x, w, a_s, a_d):
    return pl.pallas_call(
        _mm_proj_body,
        grid=(NB,),
        in_specs=[pl.BlockSpec((BN, D), lambda i: (i, 0)),
                  pl.BlockSpec((D, D), lambda i: (0, 0)),
                  pl.BlockSpec((D, HP), lambda i: (0, 0)),
                  pl.BlockSpec((D, HP), lambda i: (0, 0))],
        out_specs=[pl.BlockSpec((BN, D), lambda i: (i, 0)),
                   pl.BlockSpec((BN, HP), lambda i: (i, 0)),
                   pl.BlockSpec((BN, HP), lambda i: (i, 0))],
        out_shape=[jax.ShapeDtypeStruct((N, D), f32),
                   jax.ShapeDtypeStruct((N, HP), f32),
                   jax.ShapeDtypeStruct((N, HP), f32)],
    )(x, w, a_s, a_d)


def _ae_mm_body(ea_ref, ve_ref, o0, o1, o2, o3):
    r = jnp.dot(ea_ref[...], ve_ref[...], preferred_element_type=f32)
    o0[...] = r[:, 0:HP]
    o1[...] = r[:, HP:2 * HP]
    o2[...] = r[:, 2 * HP:3 * HP]
    o3[...] = r[:, 3 * HP:4 * HP]


def _ae_mm(edge_attr, vec):
    eout = pl.BlockSpec((BE, HP), lambda i: (i, 0))
    return pl.pallas_call(
        _ae_mm_body,
        grid=(E // BE,),
        in_specs=[pl.BlockSpec((BE, D), lambda i: (i, 0)),
                  pl.BlockSpec((D, NL * HP), lambda i: (0, 0))],
        out_specs=[eout, eout, eout, eout],
        out_shape=[jax.ShapeDtypeStruct((E, HP), f32)] * NL,
    )(edge_attr, vec)


def _dense_mid_body(p_ref, bias_ref, g_ref, be_ref, w_ref, as_ref, ad_ref,
                    h_ref, asg_ref, adg_ref, sum_ref, ssq_ref):
    ph = pl.program_id(0)
    bi = pl.program_id(1)
    o = p_ref[0] + p_ref[1] + bias_ref[...]

    @pl.when(jnp.logical_and(ph == 0, bi == 0))
    def _():
        sum_ref[...] = jnp.zeros_like(sum_ref)
        ssq_ref[...] = jnp.zeros_like(ssq_ref)

    @pl.when(ph == 0)
    def _():
        sum_ref[...] += o.sum(axis=0, keepdims=True)
        ssq_ref[...] += (o * o).sum(axis=0, keepdims=True)

    @pl.when(ph == 1)
    def _():
        mean = sum_ref[...] / N
        var = ssq_ref[...] / N - mean * mean
        xn = (o - mean) * lax.rsqrt(var + 1e-5) * g_ref[...] + be_ref[...]
        a = jnp.where(xn > 0, xn, jnp.exp(xn) - 1.0)
        h = jnp.dot(a, w_ref[...], preferred_element_type=f32)
        h_ref[...] = h
        asg_ref[...] = jnp.dot(h, as_ref[...], preferred_element_type=f32)
        adg_ref[...] = jnp.dot(h, ad_ref[...], preferred_element_type=f32)


def _dense_mid(p, bias, g, be, w, a_s, a_d):
    vec = pl.BlockSpec((1, D), lambda ph, i: (0, 0))
    return pl.pallas_call(
        _dense_mid_body,
        grid=(2, NB),
        in_specs=[pl.BlockSpec((2, BN, D), lambda ph, i: (0, i, 0)),
                  vec, vec, vec,
                  pl.BlockSpec((D, D), lambda ph, i: (0, 0)),
                  pl.BlockSpec((D, HP), lambda ph, i: (0, 0)),
                  pl.BlockSpec((D, HP), lambda ph, i: (0, 0))],
        out_specs=[pl.BlockSpec((BN, D), lambda ph, i: (i, 0)),
                   pl.BlockSpec((BN, HP), lambda ph, i: (i, 0)),
                   pl.BlockSpec((BN, HP), lambda ph, i: (i, 0))],
        out_shape=[jax.ShapeDtypeStruct((N, D), f32),
                   jax.ShapeDtypeStruct((N, HP), f32),
                   jax.ShapeDtypeStruct((N, HP), f32)],
        scratch_shapes=[pltpu.VMEM((1, D), f32), pltpu.VMEM((1, D), f32)],
    )(p, bias, g, be, w, a_s, a_d)


def _dense_final_body(p_ref, b3_ref, g3_ref, be3_ref, batch_ref,
                      out_ref, sum_ref, ssq_ref, pool_ref, cnt_ref):
    ph = pl.program_id(0)
    bi = pl.program_id(1)
    o = p_ref[0] + p_ref[1]
    om = o[:, 0:C]
    for hh in range(1, NH):
        om = om + o[:, hh * C:(hh + 1) * C]
    om = om * (1.0 / NH) + b3_ref[...]

    @pl.when(jnp.logical_and(ph == 0, bi == 0))
    def _():
        sum_ref[...] = jnp.zeros_like(sum_ref)
        ssq_ref[...] = jnp.zeros_like(ssq_ref)
        pool_ref[...] = jnp.zeros_like(pool_ref)
        cnt_ref[...] = jnp.zeros_like(cnt_ref)

    @pl.when(ph == 0)
    def _():
        sum_ref[...] += om.sum(axis=0, keepdims=True)
        ssq_ref[...] += (om * om).sum(axis=0, keepdims=True)

    @pl.when(ph == 1)
    def _():
        mean = sum_ref[...] / N
        var = ssq_ref[...] / N - mean * mean
        xn = (om - mean) * lax.rsqrt(var + 1e-5) * g3_ref[...] + be3_ref[...]
        a = jnp.where(xn > 0, xn, jnp.exp(xn) - 1.0)
        ids = lax.broadcasted_iota(jnp.int32, (BN, NG), 1)
        pf = (batch_ref[...] == ids).astype(f32)
        dn = (((0,), (0,)), ((), ()))
        pool_ref[...] += lax.dot_general(pf, a, dn, preferred_element_type=f32)
        cnt_ref[...] += lax.dot_general(pf, jnp.ones((BN, 1), f32), dn,
                                        preferred_element_type=f32)

        @pl.when(bi == NB - 1)
        def _():
            out_ref[...] = pool_ref[...] / jnp.maximum(cnt_ref[...], 1.0)


def _dense_final(p, b3, g3, be3, batch2d):
    vec = pl.BlockSpec((1, C), lambda ph, i: (0, 0))
    return pl.pallas_call(
        _dense_final_body,
        grid=(2, NB),
        in_specs=[pl.BlockSpec((2, BN, D), lambda ph, i: (0, i, 0)),
                  vec, vec, vec,
                  pl.BlockSpec((BN, 1), lambda ph, i: (i, 0))],
        out_specs=pl.BlockSpec((NG, C), lambda ph, i: (0, 0)),
        out_shape=jax.ShapeDtypeStruct((NG, C), f32),
        scratch_shapes=[pltpu.VMEM((1, C), f32), pltpu.VMEM((1, C), f32),
                        pltpu.VMEM((NG, C), f32), pltpu.VMEM((NG, 1), f32)],
    )(p, b3, g3, be3, batch2d)


def kernel(x, edge_index, edge_attr, batch, W, att_src, att_dst, W_edge,
           att_edge, bias012, bias3, gamma012, beta012, gamma3, beta3):
    src = edge_index[0]
    dst = edge_index[1]
    hc = jnp.arange(D)
    heads = jnp.arange(NH)
    blkmask = (hc[:, None] // C == heads[None, :]).astype(f32)
    zpad = jnp.zeros((NL, D, HP - NH), f32)
    a_s = jnp.concatenate(
        [blkmask[None] * att_src.reshape(NL, D, 1), zpad], axis=2)
    a_d = jnp.concatenate(
        [blkmask[None] * att_dst.reshape(NL, D, 1), zpad], axis=2)
    ve = (W_edge.reshape(NL, D, NH, C) * att_edge[:, None]).sum(-1)
    ve = jnp.concatenate([ve, jnp.zeros((NL, D, HP - NH), f32)], axis=2)
    vec = ve.transpose(1, 0, 2).reshape(D, NL * HP)

    ae_all = _ae_mm(edge_attr, vec)
    z16 = jnp.zeros((NPAD, HP), f32)
    z128 = jnp.zeros((NPAD, 128), f32)

    h, asg, adg = _mm_proj(x, W[0], a_s[0], a_d[0])
    out = None
    for i in range(NL):
        ex, dp = _sc_attn(src, dst, ae_all[i], asg, adg, z16)
        op = _sc_msg(src, dst, ex, dp[0], dp[1], h, z128)
        if i < NL - 1:
            h, asg, adg = _dense_mid(op, bias012[i].reshape(1, D),
                                     gamma012[i].reshape(1, D),
                                     beta012[i].reshape(1, D),
                                     W[i + 1], a_s[i + 1], a_d[i + 1])
        else:
            out = _dense_final(op, bias3.reshape(1, C), gamma3.reshape(1, C),
                               beta3.reshape(1, C),
                               batch.reshape(N, 1).astype(jnp.int32))
    return out
